# baseline (speedup 1.0000x reference)
import functools

import jax
import jax.numpy as jnp
from jax import lax
from jax.experimental import pallas as pl
from jax.experimental.pallas import tpu as pltpu
from jax.experimental.pallas import tpu_sc as plsc

N = 10000
E = 320000
FEAT = 128
HID = 128
CLS = 64

NC, NS = 2, 16
NT = NC * NS
CH = 128
ROWS = E // CH
RPW = 80
LROWS = ROWS - (NT - 1) * RPW
IH = 40
ARP = 624
REM_BASE = NS * ARP
REM = N - REM_BASE
ZCH = 104
RB = 2000

_mesh = plsc.VectorSubcoreMesh(core_axis_name="c", subcore_axis_name="s")
_sc_params = pltpu.CompilerParams(use_tc_tiling_on_sc=False)


DW = 16

DLAG = 8

@functools.partial(
    pl.kernel,
    out_type=jax.ShapeDtypeStruct((2 * N, DW), jnp.float32),
    mesh=_mesh,
    compiler_params=_sc_params,
    scratch_types=[
        pltpu.VMEM_SHARED((N, DW), jnp.float32),
        pltpu.VMEM((RPW, CH), jnp.int32),
        pltpu.VMEM((CH, DW), jnp.float32),
        pltpu.VMEM((ZCH, DW), jnp.float32),
        pltpu.SemaphoreType.DMA,
    ],
)
def _deg_kernel(dst2_hbm, out_hbm, acc_sh, didx, ones_v, zeros_v, sem):
    c = lax.axis_index("c")
    s = lax.axis_index("s")
    wid = c * NS + s
    rbase = s * ARP
    ones16 = jnp.ones((DW,), jnp.float32)
    zeros16 = jnp.zeros((DW,), jnp.float32)

    rowbase = wid * RPW
    nrows = jnp.where(wid == NT - 1, LROWS, RPW)

    @pl.when(wid < NT - 1)
    def _():
        pltpu.sync_copy(dst2_hbm.at[pl.ds(rowbase, RPW)], didx)

    @pl.when(wid == NT - 1)
    def _():
        pltpu.sync_copy(dst2_hbm.at[pl.ds(rowbase, LROWS)],
                        didx.at[pl.ds(0, LROWS)])

    def ob(i, carry):
        ones_v[i] = ones16
        return carry

    lax.fori_loop(0, CH, ob, 0)

    def zb(i, carry):
        zeros_v[i] = zeros16
        return carry

    lax.fori_loop(0, ZCH, zb, 0)
    for k in range(ARP // ZCH):
        pltpu.sync_copy(zeros_v, acc_sh.at[pl.ds(rbase + k * ZCH, ZCH)])

    @pl.when(s == NS - 1)
    def _():
        pltpu.sync_copy(zeros_v.at[pl.ds(0, REM)],
                        acc_sh.at[pl.ds(REM_BASE, REM)])

    plsc.subcore_barrier()

    def body(j, carry):
        pltpu.sync_copy(ones_v, acc_sh.at[didx.at[j]], add=True)
        return carry

    lax.fori_loop(0, nrows, body, 0)

    plsc.subcore_barrier()
    pltpu.sync_copy(acc_sh.at[pl.ds(rbase, ARP)],
                    out_hbm.at[pl.ds(c * N + rbase, ARP)])

    @pl.when(s == NS - 1)
    def _():
        pltpu.sync_copy(acc_sh.at[pl.ds(REM_BASE, REM)],
                        out_hbm.at[pl.ds(c * N + REM_BASE, REM)])


def _make_agg(F):
    @functools.partial(
        pl.kernel,
        out_type=jax.ShapeDtypeStruct((2 * N, F), jnp.float32),
        mesh=_mesh,
        compiler_params=_sc_params,
        scratch_types=[
            pltpu.VMEM_SHARED((N, F), jnp.float32),
            pltpu.VMEM((CH,), jnp.int32),
            pltpu.VMEM((CH,), jnp.int32),
            pltpu.VMEM((RPW, CH), jnp.int32),
            pltpu.VMEM((CH, F), jnp.float32),
            pltpu.VMEM((CH, F), jnp.float32),
            pltpu.SemaphoreType.DMA,
            pltpu.SemaphoreType.DMA,
            pltpu.SemaphoreType.DMA,
        ],
    )
    def agg(g_hbm, src_hbm, dst2_hbm, out_hbm,
            acc_sh, sidxa, sidxb, didx, buf0, buf1, sem, sem2, sem3):
        c = lax.axis_index("c")
        s = lax.axis_index("s")
        wid = c * NS + s
        rbase = s * ARP
        rowbase = wid * RPW
        nrows = jnp.where(wid == NT - 1, LROWS, RPW)

        @pl.when(c == 0)
        def _():
            pltpu.sync_copy(g_hbm.at[pl.ds(rbase, ARP)],
                            acc_sh.at[pl.ds(rbase, ARP)])

            @pl.when(s == NS - 1)
            def _():
                pltpu.sync_copy(g_hbm.at[pl.ds(REM_BASE, REM)],
                                acc_sh.at[pl.ds(REM_BASE, REM)])

        @pl.when(c != 0)
        def _():
            zeros16 = jnp.zeros((16,), jnp.float32)

            def zb(i, carry):
                r = i // (F // 16)
                k = i % (F // 16)
                buf0[r, pl.ds(k * 16, 16)] = zeros16
                return carry

            lax.fori_loop(0, CH * F // 16, zb, 0)
            for k in range(ARP // CH):
                pltpu.sync_copy(buf0, acc_sh.at[pl.ds(rbase + k * CH, CH)])
            pltpu.sync_copy(buf0.at[pl.ds(0, ARP - (ARP // CH) * CH)],
                            acc_sh.at[pl.ds(rbase + (ARP // CH) * CH,
                                            ARP - (ARP // CH) * CH)])

            @pl.when(s == NS - 1)
            def _():
                pltpu.sync_copy(buf0.at[pl.ds(0, REM)],
                                acc_sh.at[pl.ds(REM_BASE, REM)])

        plsc.subcore_barrier()

        @pl.when(wid == NT - 1)
        def _():
            pltpu.sync_copy(dst2_hbm.at[pl.ds(rowbase, LROWS)],
                            didx.at[pl.ds(0, LROWS)])

        @pl.when(wid < NT - 1)
        def _():
            pltpu.sync_copy(dst2_hbm.at[pl.ds(rowbase, RPW)], didx)

        ebase = rowbase * CH
        pltpu.sync_copy(src_hbm.at[pl.ds(ebase, CH)], sidxa)
        pltpu.sync_copy(src_hbm.at[pl.ds(ebase + CH, CH)], sidxb)
        pltpu.async_copy(g_hbm.at[sidxa], buf0, sem).wait()

        def body(k, carry):
            j0 = 2 * k
            hb = pltpu.async_copy(g_hbm.at[sidxb], buf1, sem)
            s0 = pltpu.async_copy(buf0, acc_sh.at[didx.at[j0]], sem3,
                                  add=True)

            @pl.when(j0 + 2 < nrows)
            def _():
                ra = pltpu.async_copy(
                    src_hbm.at[pl.ds(ebase + (j0 + 2) * CH, CH)],
                    sidxa, sem2)
                hb.wait()
                ra.wait()
                s0.wait()
                ha = pltpu.async_copy(g_hbm.at[sidxa], buf0, sem)
                s1 = pltpu.async_copy(buf1, acc_sh.at[didx.at[j0 + 1]],
                                      sem3, add=True)

                @pl.when(j0 + 3 < nrows)
                def _():
                    pltpu.async_copy(
                        src_hbm.at[pl.ds(ebase + (j0 + 3) * CH, CH)],
                        sidxb, sem2).wait()

                ha.wait()
                s1.wait()

            @pl.when(j0 + 2 >= nrows)
            def _():
                hb.wait()
                s0.wait()
                pltpu.sync_copy(buf1, acc_sh.at[didx.at[j0 + 1]], add=True)

            return carry

        lax.fori_loop(0, nrows // 2, body, 0)

        plsc.subcore_barrier()
        pltpu.sync_copy(acc_sh.at[pl.ds(rbase, ARP)],
                        out_hbm.at[pl.ds(c * N + rbase, ARP)])

        @pl.when(s == NS - 1)
        def _():
            pltpu.sync_copy(acc_sh.at[pl.ds(REM_BASE, REM)],
                            out_hbm.at[pl.ds(c * N + REM_BASE, REM)])

    return agg


_agg128 = _make_agg(HID)
_agg64 = _make_agg(CLS)


RPW2 = 160
LROWS2 = ROWS - (NS - 1) * RPW2

def _make_agg_cs(F):
    FH = F // 2

    @functools.partial(
        pl.kernel,
        out_type=jax.ShapeDtypeStruct((N, F), jnp.float32),
        mesh=_mesh,
        compiler_params=_sc_params,
        scratch_types=[
            pltpu.VMEM_SHARED((N, FH), jnp.float32),
            pltpu.VMEM_SHARED((N, FH), jnp.float32),
            pltpu.VMEM((CH,), jnp.int32),
            pltpu.VMEM((CH,), jnp.int32),
            pltpu.VMEM((RPW2, CH), jnp.int32),
            pltpu.VMEM((CH, FH), jnp.float32),
            pltpu.VMEM((CH, FH), jnp.float32),
            pltpu.SemaphoreType.DMA,
            pltpu.SemaphoreType.DMA,
            pltpu.SemaphoreType.DMA,
        ],
    )
    def agg(g_hbm, src_hbm, dst2_hbm, out_hbm,
            gtab_sh, acc_sh, sidxa, sidxb, didx, buf0, buf1,
            sem, sem2, sem3):
        c = lax.axis_index("c")
        s = lax.axis_index("s")
        rbase = s * ARP
        colbase = c * FH

        pltpu.sync_copy(g_hbm.at[pl.ds(rbase, ARP), pl.ds(colbase, FH)],
                        gtab_sh.at[pl.ds(rbase, ARP)])
        pltpu.sync_copy(g_hbm.at[pl.ds(rbase, ARP), pl.ds(colbase, FH)],
                        acc_sh.at[pl.ds(rbase, ARP)])

        @pl.when(s == NS - 1)
        def _():
            pltpu.sync_copy(
                g_hbm.at[pl.ds(REM_BASE, REM), pl.ds(colbase, FH)],
                gtab_sh.at[pl.ds(REM_BASE, REM)])
            pltpu.sync_copy(
                g_hbm.at[pl.ds(REM_BASE, REM), pl.ds(colbase, FH)],
                acc_sh.at[pl.ds(REM_BASE, REM)])

        rowbase = s * RPW2
        nrows = jnp.where(s == NS - 1, LROWS2, RPW2)

        @pl.when(s == NS - 1)
        def _():
            pltpu.sync_copy(dst2_hbm.at[pl.ds(rowbase, LROWS2)],
                            didx.at[pl.ds(0, LROWS2)])

        @pl.when(s < NS - 1)
        def _():
            pltpu.sync_copy(dst2_hbm.at[pl.ds(rowbase, RPW2)], didx)

        plsc.subcore_barrier()

        ebase = rowbase * CH
        pltpu.sync_copy(src_hbm.at[pl.ds(ebase, CH)], sidxa)
        pltpu.sync_copy(src_hbm.at[pl.ds(ebase + CH, CH)], sidxb)
        pltpu.async_copy(gtab_sh.at[sidxa], buf0, sem).wait()

        def body(k, carry):
            j0 = 2 * k
            hb = pltpu.async_copy(gtab_sh.at[sidxb], buf1, sem)
            s0 = pltpu.async_copy(buf0, acc_sh.at[didx.at[j0]], sem3,
                                  add=True)

            @pl.when(j0 + 2 < nrows)
            def _():
                ra = pltpu.async_copy(
                    src_hbm.at[pl.ds(ebase + (j0 + 2) * CH, CH)],
                    sidxa, sem2)
                hb.wait()
                ra.wait()
                s0.wait()
                ha = pltpu.async_copy(gtab_sh.at[sidxa], buf0, sem)
                s1 = pltpu.async_copy(buf1, acc_sh.at[didx.at[j0 + 1]],
                                      sem3, add=True)

                @pl.when(j0 + 3 < nrows)
                def _():
                    pltpu.async_copy(
                        src_hbm.at[pl.ds(ebase + (j0 + 3) * CH, CH)],
                        sidxb, sem2).wait()

                ha.wait()
                s1.wait()

            @pl.when(j0 + 2 >= nrows)
            def _():
                hb.wait()
                s0.wait()
                pltpu.sync_copy(buf1, acc_sh.at[didx.at[j0 + 1]], add=True)

            return carry

        lax.fori_loop(0, nrows // 2, body, 0)

        plsc.subcore_barrier()
        pltpu.sync_copy(acc_sh.at[pl.ds(rbase, ARP)],
                        out_hbm.at[pl.ds(rbase, ARP), pl.ds(colbase, FH)])

        @pl.when(s == NS - 1)
        def _():
            pltpu.sync_copy(
                acc_sh.at[pl.ds(REM_BASE, REM)],
                out_hbm.at[pl.ds(REM_BASE, REM), pl.ds(colbase, FH)])

    return agg


_aggcs128 = _make_agg_cs(HID)
_aggcs64 = _make_agg_cs(CLS)


def _g1_body(d0_ref, d1_ref, x_ref, w_ref, g_ref, dinv_ref):
    deg = (d0_ref[...] + d1_ref[...])[:, 0:1] + 1.0
    dinv = lax.rsqrt(deg)
    h = jnp.dot(jnp.maximum(x_ref[...], 0.0), w_ref[...],
                preferred_element_type=jnp.float32)
    g_ref[...] = dinv * h
    dinv_ref[...] = dinv


def _tc_g1(d0, d1, x, w1):
    return pl.pallas_call(
        _g1_body,
        grid=(N // RB,),
        in_specs=[
            pl.BlockSpec((RB, DW), lambda i: (i, 0)),
            pl.BlockSpec((RB, DW), lambda i: (i, 0)),
            pl.BlockSpec((RB, FEAT), lambda i: (i, 0)),
            pl.BlockSpec((FEAT, HID), lambda i: (0, 0)),
        ],
        out_specs=[
            pl.BlockSpec((RB, HID), lambda i: (i, 0)),
            pl.BlockSpec((RB, 1), lambda i: (i, 0)),
        ],
        out_shape=[
            jax.ShapeDtypeStruct((N, HID), jnp.float32),
            jax.ShapeDtypeStruct((N, 1), jnp.float32),
        ],
    )(d0, d1, x, w1)


def _g2_body(a0_ref, a1_ref, dinv_ref, b1_ref, w_ref, g_ref):
    dinv = dinv_ref[...]
    t = dinv * (a0_ref[...] + a1_ref[...]) + b1_ref[...]
    h = jnp.maximum(t, 0.0)
    g_ref[...] = dinv * jnp.dot(h, w_ref[...],
                                preferred_element_type=jnp.float32)


def _tc_g2(a0, a1, dinv, b1, w2):
    return pl.pallas_call(
        _g2_body,
        grid=(N // RB,),
        in_specs=[
            pl.BlockSpec((RB, HID), lambda i: (i, 0)),
            pl.BlockSpec((RB, HID), lambda i: (i, 0)),
            pl.BlockSpec((RB, 1), lambda i: (i, 0)),
            pl.BlockSpec((1, HID), lambda i: (0, 0)),
            pl.BlockSpec((HID, CLS), lambda i: (0, 0)),
        ],
        out_specs=pl.BlockSpec((RB, CLS), lambda i: (i, 0)),
        out_shape=jax.ShapeDtypeStruct((N, CLS), jnp.float32),
    )(a0, a1, dinv, b1, w2)


def _g2s_body(a_ref, dinv_ref, b1_ref, w_ref, g_ref):
    dinv = dinv_ref[...]
    t = dinv * a_ref[...] + b1_ref[...]
    h = jnp.maximum(t, 0.0)
    g_ref[...] = dinv * jnp.dot(h, w_ref[...],
                                preferred_element_type=jnp.float32)


def _tc_g2s(a, dinv, b1, w2):
    return pl.pallas_call(
        _g2s_body,
        grid=(N // RB,),
        in_specs=[
            pl.BlockSpec((RB, HID), lambda i: (i, 0)),
            pl.BlockSpec((RB, 1), lambda i: (i, 0)),
            pl.BlockSpec((1, HID), lambda i: (0, 0)),
            pl.BlockSpec((HID, CLS), lambda i: (0, 0)),
        ],
        out_specs=pl.BlockSpec((RB, CLS), lambda i: (i, 0)),
        out_shape=jax.ShapeDtypeStruct((N, CLS), jnp.float32),
    )(a, dinv, b1, w2)


def _outs_body(a_ref, dinv_ref, b2_ref, o_ref):
    o_ref[...] = dinv_ref[...] * a_ref[...] + b2_ref[...]


def _tc_outs(a, dinv, b2):
    return pl.pallas_call(
        _outs_body,
        grid=(N // RB,),
        in_specs=[
            pl.BlockSpec((RB, CLS), lambda i: (i, 0)),
            pl.BlockSpec((RB, 1), lambda i: (i, 0)),
            pl.BlockSpec((1, CLS), lambda i: (0, 0)),
        ],
        out_specs=pl.BlockSpec((RB, CLS), lambda i: (i, 0)),
        out_shape=jax.ShapeDtypeStruct((N, CLS), jnp.float32),
    )(a, dinv, b2)


def _out_body(a0_ref, a1_ref, dinv_ref, b2_ref, o_ref):
    o_ref[...] = (dinv_ref[...] * (a0_ref[...] + a1_ref[...])
                  + b2_ref[...])


def _tc_out(a0, a1, dinv, b2):
    return pl.pallas_call(
        _out_body,
        grid=(N // RB,),
        in_specs=[
            pl.BlockSpec((RB, CLS), lambda i: (i, 0)),
            pl.BlockSpec((RB, CLS), lambda i: (i, 0)),
            pl.BlockSpec((RB, 1), lambda i: (i, 0)),
            pl.BlockSpec((1, CLS), lambda i: (0, 0)),
        ],
        out_specs=pl.BlockSpec((RB, CLS), lambda i: (i, 0)),
        out_shape=jax.ShapeDtypeStruct((N, CLS), jnp.float32),
    )(a0, a1, dinv, b2)


def kernel(x, edge_index, W1, b1, W2, b2):
    src1 = edge_index[0]
    dst2 = edge_index[1].reshape(ROWS, CH)

    degp = _deg_kernel(dst2)
    g1, dinv = _tc_g1(degp[:N], degp[N:], x, W1)

    agg1 = _agg128(g1, src1, dst2)
    g2 = _tc_g2(agg1[:N], agg1[N:], dinv, b1.reshape(1, HID), W2)

    agg2 = _agg64(g2, src1, dst2)
    return _tc_out(agg2[:N], agg2[N:], dinv, b2.reshape(1, CLS))


# speedup vs baseline: 1.0051x; 1.0051x over previous
import functools

import jax
import jax.numpy as jnp
from jax import lax
from jax.experimental import pallas as pl
from jax.experimental.pallas import tpu as pltpu
from jax.experimental.pallas import tpu_sc as plsc

N = 10000
E = 320000
FEAT = 128
HID = 128
CLS = 64

NC, NS = 2, 16
NT = NC * NS
CH = 128
ROWS = E // CH
RPW = 80
LROWS = ROWS - (NT - 1) * RPW
IH = 40
ARP = 624
REM_BASE = NS * ARP
REM = N - REM_BASE
ZCH = 104
RB = 2000

_mesh = plsc.VectorSubcoreMesh(core_axis_name="c", subcore_axis_name="s")
_sc_params = pltpu.CompilerParams(use_tc_tiling_on_sc=False)


DW = 16

DLAG = 8

@functools.partial(
    pl.kernel,
    out_type=jax.ShapeDtypeStruct((2 * N, DW), jnp.float32),
    mesh=_mesh,
    compiler_params=_sc_params,
    scratch_types=[
        pltpu.VMEM_SHARED((N, DW), jnp.float32),
        pltpu.VMEM((RPW, CH), jnp.int32),
        pltpu.VMEM((CH, DW), jnp.float32),
        pltpu.VMEM((ZCH, DW), jnp.float32),
        pltpu.SemaphoreType.DMA,
    ],
)
def _deg_kernel(dst2_hbm, out_hbm, acc_sh, didx, ones_v, zeros_v, sem):
    c = lax.axis_index("c")
    s = lax.axis_index("s")
    wid = c * NS + s
    rbase = s * ARP
    ones16 = jnp.ones((DW,), jnp.float32)
    zeros16 = jnp.zeros((DW,), jnp.float32)

    rowbase = wid * RPW
    nrows = jnp.where(wid == NT - 1, LROWS, RPW)

    @pl.when(wid < NT - 1)
    def _():
        pltpu.sync_copy(dst2_hbm.at[pl.ds(rowbase, RPW)], didx)

    @pl.when(wid == NT - 1)
    def _():
        pltpu.sync_copy(dst2_hbm.at[pl.ds(rowbase, LROWS)],
                        didx.at[pl.ds(0, LROWS)])

    def ob(i, carry):
        ones_v[i] = ones16
        return carry

    lax.fori_loop(0, CH, ob, 0)

    def zb(i, carry):
        zeros_v[i] = zeros16
        return carry

    lax.fori_loop(0, ZCH, zb, 0)
    for k in range(ARP // ZCH):
        pltpu.sync_copy(zeros_v, acc_sh.at[pl.ds(rbase + k * ZCH, ZCH)])

    @pl.when(s == NS - 1)
    def _():
        pltpu.sync_copy(zeros_v.at[pl.ds(0, REM)],
                        acc_sh.at[pl.ds(REM_BASE, REM)])

    plsc.subcore_barrier()

    def body(j, carry):
        pltpu.sync_copy(ones_v, acc_sh.at[didx.at[j]], add=True)
        return carry

    lax.fori_loop(0, nrows, body, 0)

    plsc.subcore_barrier()
    pltpu.sync_copy(acc_sh.at[pl.ds(rbase, ARP)],
                    out_hbm.at[pl.ds(c * N + rbase, ARP)])

    @pl.when(s == NS - 1)
    def _():
        pltpu.sync_copy(acc_sh.at[pl.ds(REM_BASE, REM)],
                        out_hbm.at[pl.ds(c * N + REM_BASE, REM)])


def _make_agg(F):
    @functools.partial(
        pl.kernel,
        out_type=jax.ShapeDtypeStruct((2 * N, F), jnp.float32),
        mesh=_mesh,
        compiler_params=_sc_params,
        scratch_types=[
            pltpu.VMEM_SHARED((N, F), jnp.float32),
            pltpu.VMEM((CH,), jnp.int32),
            pltpu.VMEM((CH,), jnp.int32),
            pltpu.VMEM((RPW, CH), jnp.int32),
            pltpu.VMEM((CH, F), jnp.float32),
            pltpu.VMEM((CH, F), jnp.float32),
            pltpu.SemaphoreType.DMA,
            pltpu.SemaphoreType.DMA,
            pltpu.SemaphoreType.DMA,
        ],
    )
    def agg(g_hbm, src_hbm, dst2_hbm, out_hbm,
            acc_sh, sidxa, sidxb, didx, buf0, buf1, sem, sem2, sem3):
        c = lax.axis_index("c")
        s = lax.axis_index("s")
        wid = c * NS + s
        rbase = s * ARP
        rowbase = wid * RPW
        nrows = jnp.where(wid == NT - 1, LROWS, RPW)

        @pl.when(c == 0)
        def _():
            pltpu.sync_copy(g_hbm.at[pl.ds(rbase, ARP)],
                            acc_sh.at[pl.ds(rbase, ARP)])

            @pl.when(s == NS - 1)
            def _():
                pltpu.sync_copy(g_hbm.at[pl.ds(REM_BASE, REM)],
                                acc_sh.at[pl.ds(REM_BASE, REM)])

        @pl.when(c != 0)
        def _():
            zeros16 = jnp.zeros((16,), jnp.float32)

            def zb(i, carry):
                r = i // (F // 16)
                k = i % (F // 16)
                buf0[r, pl.ds(k * 16, 16)] = zeros16
                return carry

            lax.fori_loop(0, CH * F // 16, zb, 0)
            for k in range(ARP // CH):
                pltpu.sync_copy(buf0, acc_sh.at[pl.ds(rbase + k * CH, CH)])
            pltpu.sync_copy(buf0.at[pl.ds(0, ARP - (ARP // CH) * CH)],
                            acc_sh.at[pl.ds(rbase + (ARP // CH) * CH,
                                            ARP - (ARP // CH) * CH)])

            @pl.when(s == NS - 1)
            def _():
                pltpu.sync_copy(buf0.at[pl.ds(0, REM)],
                                acc_sh.at[pl.ds(REM_BASE, REM)])

        plsc.subcore_barrier()

        @pl.when(wid == NT - 1)
        def _():
            pltpu.sync_copy(dst2_hbm.at[pl.ds(rowbase, LROWS)],
                            didx.at[pl.ds(0, LROWS)])

        @pl.when(wid < NT - 1)
        def _():
            pltpu.sync_copy(dst2_hbm.at[pl.ds(rowbase, RPW)], didx)

        ebase = rowbase * CH
        pltpu.sync_copy(src_hbm.at[pl.ds(ebase, CH)], sidxa)
        pltpu.sync_copy(src_hbm.at[pl.ds(ebase + CH, CH)], sidxb)
        pltpu.async_copy(g_hbm.at[sidxa], buf0, sem).wait()

        def body(k, carry):
            j0 = 2 * k
            hb = pltpu.async_copy(g_hbm.at[sidxb], buf1, sem)
            s0 = pltpu.async_copy(buf0, acc_sh.at[didx.at[j0]], sem3,
                                  add=True)

            @pl.when(j0 + 2 < nrows)
            def _():
                ra = pltpu.async_copy(
                    src_hbm.at[pl.ds(ebase + (j0 + 2) * CH, CH)],
                    sidxa, sem2)
                hb.wait()
                ra.wait()
                s0.wait()
                ha = pltpu.async_copy(g_hbm.at[sidxa], buf0, sem)
                s1 = pltpu.async_copy(buf1, acc_sh.at[didx.at[j0 + 1]],
                                      sem3, add=True)

                @pl.when(j0 + 3 < nrows)
                def _():
                    pltpu.async_copy(
                        src_hbm.at[pl.ds(ebase + (j0 + 3) * CH, CH)],
                        sidxb, sem2).wait()

                ha.wait()
                s1.wait()

            @pl.when(j0 + 2 >= nrows)
            def _():
                hb.wait()
                s0.wait()
                pltpu.sync_copy(buf1, acc_sh.at[didx.at[j0 + 1]], add=True)

            return carry

        lax.fori_loop(0, nrows // 2, body, 0)

        plsc.subcore_barrier()
        pltpu.sync_copy(acc_sh.at[pl.ds(rbase, ARP)],
                        out_hbm.at[pl.ds(c * N + rbase, ARP)])

        @pl.when(s == NS - 1)
        def _():
            pltpu.sync_copy(acc_sh.at[pl.ds(REM_BASE, REM)],
                            out_hbm.at[pl.ds(c * N + REM_BASE, REM)])

    return agg


_agg128 = _make_agg(HID)
_agg64 = _make_agg(CLS)


RPW2 = 160
LROWS2 = ROWS - (NS - 1) * RPW2

def _make_agg_cs(F):
    FH = F // 2

    @functools.partial(
        pl.kernel,
        out_type=jax.ShapeDtypeStruct((N, F), jnp.float32),
        mesh=_mesh,
        compiler_params=_sc_params,
        scratch_types=[
            pltpu.VMEM_SHARED((N, FH), jnp.float32),
            pltpu.VMEM_SHARED((N, FH), jnp.float32),
            pltpu.VMEM((CH,), jnp.int32),
            pltpu.VMEM((CH,), jnp.int32),
            pltpu.VMEM((RPW2, CH), jnp.int32),
            pltpu.VMEM((CH, FH), jnp.float32),
            pltpu.VMEM((CH, FH), jnp.float32),
            pltpu.SemaphoreType.DMA,
            pltpu.SemaphoreType.DMA,
            pltpu.SemaphoreType.DMA,
        ],
    )
    def agg(g_hbm, src_hbm, dst2_hbm, out_hbm,
            gtab_sh, acc_sh, sidxa, sidxb, didx, buf0, buf1,
            sem, sem2, sem3):
        c = lax.axis_index("c")
        s = lax.axis_index("s")
        rbase = s * ARP
        colbase = c * FH

        pltpu.sync_copy(g_hbm.at[pl.ds(rbase, ARP), pl.ds(colbase, FH)],
                        gtab_sh.at[pl.ds(rbase, ARP)])
        pltpu.sync_copy(g_hbm.at[pl.ds(rbase, ARP), pl.ds(colbase, FH)],
                        acc_sh.at[pl.ds(rbase, ARP)])

        @pl.when(s == NS - 1)
        def _():
            pltpu.sync_copy(
                g_hbm.at[pl.ds(REM_BASE, REM), pl.ds(colbase, FH)],
                gtab_sh.at[pl.ds(REM_BASE, REM)])
            pltpu.sync_copy(
                g_hbm.at[pl.ds(REM_BASE, REM), pl.ds(colbase, FH)],
                acc_sh.at[pl.ds(REM_BASE, REM)])

        rowbase = s * RPW2
        nrows = jnp.where(s == NS - 1, LROWS2, RPW2)

        @pl.when(s == NS - 1)
        def _():
            pltpu.sync_copy(dst2_hbm.at[pl.ds(rowbase, LROWS2)],
                            didx.at[pl.ds(0, LROWS2)])

        @pl.when(s < NS - 1)
        def _():
            pltpu.sync_copy(dst2_hbm.at[pl.ds(rowbase, RPW2)], didx)

        plsc.subcore_barrier()

        ebase = rowbase * CH
        pltpu.sync_copy(src_hbm.at[pl.ds(ebase, CH)], sidxa)
        pltpu.sync_copy(src_hbm.at[pl.ds(ebase + CH, CH)], sidxb)
        pltpu.async_copy(gtab_sh.at[sidxa], buf0, sem).wait()

        def body(k, carry):
            j0 = 2 * k
            hb = pltpu.async_copy(gtab_sh.at[sidxb], buf1, sem)
            s0 = pltpu.async_copy(buf0, acc_sh.at[didx.at[j0]], sem3,
                                  add=True)

            @pl.when(j0 + 2 < nrows)
            def _():
                ra = pltpu.async_copy(
                    src_hbm.at[pl.ds(ebase + (j0 + 2) * CH, CH)],
                    sidxa, sem2)
                hb.wait()
                ra.wait()
                s0.wait()
                ha = pltpu.async_copy(gtab_sh.at[sidxa], buf0, sem)
                s1 = pltpu.async_copy(buf1, acc_sh.at[didx.at[j0 + 1]],
                                      sem3, add=True)

                @pl.when(j0 + 3 < nrows)
                def _():
                    pltpu.async_copy(
                        src_hbm.at[pl.ds(ebase + (j0 + 3) * CH, CH)],
                        sidxb, sem2).wait()

                ha.wait()
                s1.wait()

            @pl.when(j0 + 2 >= nrows)
            def _():
                hb.wait()
                s0.wait()
                pltpu.sync_copy(buf1, acc_sh.at[didx.at[j0 + 1]], add=True)

            return carry

        lax.fori_loop(0, nrows // 2, body, 0)

        plsc.subcore_barrier()
        pltpu.sync_copy(acc_sh.at[pl.ds(rbase, ARP)],
                        out_hbm.at[pl.ds(rbase, ARP), pl.ds(colbase, FH)])

        @pl.when(s == NS - 1)
        def _():
            pltpu.sync_copy(
                acc_sh.at[pl.ds(REM_BASE, REM)],
                out_hbm.at[pl.ds(REM_BASE, REM), pl.ds(colbase, FH)])

    return agg


_aggcs128 = _make_agg_cs(HID)
_aggcs64 = _make_agg_cs(CLS)


def _g1_body(d0_ref, d1_ref, x_ref, w_ref, g_ref, dinv_ref):
    deg = (d0_ref[...] + d1_ref[...])[:, 0:1] + 1.0
    dinv = lax.rsqrt(deg)
    h = jnp.dot(jnp.maximum(x_ref[...], 0.0), w_ref[...],
                preferred_element_type=jnp.float32)
    g_ref[...] = dinv * h
    dinv_ref[...] = dinv


def _tc_g1(d0, d1, x, w1):
    return pl.pallas_call(
        _g1_body,
        grid=(N // RB,),
        in_specs=[
            pl.BlockSpec((RB, DW), lambda i: (i, 0)),
            pl.BlockSpec((RB, DW), lambda i: (i, 0)),
            pl.BlockSpec((RB, FEAT), lambda i: (i, 0)),
            pl.BlockSpec((FEAT, HID), lambda i: (0, 0)),
        ],
        out_specs=[
            pl.BlockSpec((RB, HID), lambda i: (i, 0)),
            pl.BlockSpec((RB, 1), lambda i: (i, 0)),
        ],
        out_shape=[
            jax.ShapeDtypeStruct((N, HID), jnp.float32),
            jax.ShapeDtypeStruct((N, 1), jnp.float32),
        ],
    )(d0, d1, x, w1)


def _g2_body(a0_ref, a1_ref, dinv_ref, b1_ref, w_ref, g_ref):
    dinv = dinv_ref[...]
    t = dinv * (a0_ref[...] + a1_ref[...]) + b1_ref[...]
    h = jnp.maximum(t, 0.0)
    g_ref[...] = dinv * jnp.dot(h, w_ref[...],
                                preferred_element_type=jnp.float32)


def _tc_g2(a0, a1, dinv, b1, w2):
    return pl.pallas_call(
        _g2_body,
        grid=(N // RB,),
        in_specs=[
            pl.BlockSpec((RB, HID), lambda i: (i, 0)),
            pl.BlockSpec((RB, HID), lambda i: (i, 0)),
            pl.BlockSpec((RB, 1), lambda i: (i, 0)),
            pl.BlockSpec((1, HID), lambda i: (0, 0)),
            pl.BlockSpec((HID, CLS), lambda i: (0, 0)),
        ],
        out_specs=pl.BlockSpec((RB, CLS), lambda i: (i, 0)),
        out_shape=jax.ShapeDtypeStruct((N, CLS), jnp.float32),
    )(a0, a1, dinv, b1, w2)


def _g2s_body(a_ref, dinv_ref, b1_ref, w_ref, g_ref):
    dinv = dinv_ref[...]
    t = dinv * a_ref[...] + b1_ref[...]
    h = jnp.maximum(t, 0.0)
    g_ref[...] = dinv * jnp.dot(h, w_ref[...],
                                preferred_element_type=jnp.float32)


def _tc_g2s(a, dinv, b1, w2):
    return pl.pallas_call(
        _g2s_body,
        grid=(N // RB,),
        in_specs=[
            pl.BlockSpec((RB, HID), lambda i: (i, 0)),
            pl.BlockSpec((RB, 1), lambda i: (i, 0)),
            pl.BlockSpec((1, HID), lambda i: (0, 0)),
            pl.BlockSpec((HID, CLS), lambda i: (0, 0)),
        ],
        out_specs=pl.BlockSpec((RB, CLS), lambda i: (i, 0)),
        out_shape=jax.ShapeDtypeStruct((N, CLS), jnp.float32),
    )(a, dinv, b1, w2)


def _outs_body(a_ref, dinv_ref, b2_ref, o_ref):
    o_ref[...] = dinv_ref[...] * a_ref[...] + b2_ref[...]


def _tc_outs(a, dinv, b2):
    return pl.pallas_call(
        _outs_body,
        grid=(N // RB,),
        in_specs=[
            pl.BlockSpec((RB, CLS), lambda i: (i, 0)),
            pl.BlockSpec((RB, 1), lambda i: (i, 0)),
            pl.BlockSpec((1, CLS), lambda i: (0, 0)),
        ],
        out_specs=pl.BlockSpec((RB, CLS), lambda i: (i, 0)),
        out_shape=jax.ShapeDtypeStruct((N, CLS), jnp.float32),
    )(a, dinv, b2)


def _out_body(a0_ref, a1_ref, dinv_ref, b2_ref, o_ref):
    o_ref[...] = (dinv_ref[...] * (a0_ref[...] + a1_ref[...])
                  + b2_ref[...])


def _tc_out(a0, a1, dinv, b2):
    return pl.pallas_call(
        _out_body,
        grid=(N // RB,),
        in_specs=[
            pl.BlockSpec((RB, CLS), lambda i: (i, 0)),
            pl.BlockSpec((RB, CLS), lambda i: (i, 0)),
            pl.BlockSpec((RB, 1), lambda i: (i, 0)),
            pl.BlockSpec((1, CLS), lambda i: (0, 0)),
        ],
        out_specs=pl.BlockSpec((RB, CLS), lambda i: (i, 0)),
        out_shape=jax.ShapeDtypeStruct((N, CLS), jnp.float32),
    )(a0, a1, dinv, b2)


def kernel(x, edge_index, W1, b1, W2, b2):
    src1 = edge_index[0]
    dst2 = edge_index[1].reshape(ROWS, CH)

    degp = _deg_kernel(dst2)
    g1, dinv = _tc_g1(degp[:N], degp[N:], x, W1)

    agg1 = _aggcs128(g1, src1, dst2)
    g2 = _tc_g2s(agg1, dinv, b1.reshape(1, HID), W2)

    agg2 = _aggcs64(g2, src1, dst2)
    return _tc_outs(agg2, dinv, b2.reshape(1, CLS))
